# Initial kernel scaffold; baseline (speedup 1.0000x reference)
import jax
import jax.numpy as jnp
from jax.experimental import pallas as pl


def kernel(x, edge_index, edge_weights, W1, b1, W2, b2):
    raise NotImplementedError("write your pallas kernel here")


# speedup vs baseline: 8.6082x; 8.6082x over previous
import functools

import jax
import jax.numpy as jnp
from jax import lax
from jax.experimental import pallas as pl
from jax.experimental.pallas import tpu as pltpu
from jax.experimental.pallas import tpu_sc as plsc

_N = 10000
_E = 320000
_D = 128
_H = 128

_NC = 2
_NS = 16
_NW = _NC * _NS

_K = 128
_NCHUNK = 79
_EPT = _K * _NCHUNK
_EPAD = _EPT * _NW

_NP = 10240
_RPT = _NP // _NS
_ZROWS = 128

_DEGP = 10240
_DPT = _DEGP // _NS

_ROWBLK = 1000
_GRID = _N // _ROWBLK

_mesh = plsc.VectorSubcoreMesh(core_axis_name="c", subcore_axis_name="s")


@functools.partial(
    pl.kernel,
    out_type=jax.ShapeDtypeStruct((_NC, _DEGP), jnp.float32),
    mesh=_mesh,
    scratch_types=[
        pltpu.VMEM_SHARED((_DEGP,), jnp.float32),
        pltpu.VMEM((_K,), jnp.int32),
        pltpu.VMEM((_K,), jnp.float32),
        pltpu.VMEM((_DPT,), jnp.float32),
    ],
)
def _sc_degree(c_hbm, ew_hbm, out_hbm, deg_sh, c_v, ew_v, zbuf):
    cid = lax.axis_index("c")
    sid = lax.axis_index("s")
    wid = sid * _NC + cid

    for i in range(_DPT // 16):
        zbuf[pl.ds(i * 16, 16)] = jnp.zeros((16,), jnp.float32)
    pltpu.sync_copy(zbuf, deg_sh.at[pl.ds(sid * _DPT, _DPT)])
    plsc.subcore_barrier()

    base = wid * _EPT

    def chunk(i, carry):
        off = base + i * _K
        pltpu.sync_copy(c_hbm.at[pl.ds(off, _K)], c_v)
        pltpu.sync_copy(ew_hbm.at[pl.ds(off, _K)], ew_v)
        pltpu.sync_copy(ew_v, deg_sh.at[c_v], add=True)
        return carry

    lax.fori_loop(0, _NCHUNK, chunk, 0)
    plsc.subcore_barrier()
    pltpu.sync_copy(deg_sh.at[pl.ds(sid * _DPT, _DPT)],
                    out_hbm.at[cid, pl.ds(sid * _DPT, _DPT)])


@functools.partial(
    pl.kernel,
    out_type=jax.ShapeDtypeStruct((_NC, _NP, _H), jnp.float32),
    mesh=_mesh,
    scratch_types=[
        pltpu.VMEM_SHARED((_NP, _H), jnp.float32),
        pltpu.VMEM((_K,), jnp.int32),
        pltpu.VMEM((_K,), jnp.int32),
        pltpu.VMEM((_K,), jnp.float32),
        pltpu.VMEM((_K, _H), jnp.float32),
        pltpu.VMEM((_ZROWS, _H), jnp.float32),
        pltpu.SemaphoreType.DMA,
    ],
)
def _sc_message(g_hbm, r_hbm, c_hbm, ew_hbm, out_hbm,
                acc, r_v, c_v, ew_v, rows, zbuf, sem):
    cid = lax.axis_index("c")
    sid = lax.axis_index("s")
    wid = sid * _NC + cid

    def zrow(i, carry):
        for q in range(_H // 16):
            zbuf[i, pl.ds(q * 16, 16)] = jnp.zeros((16,), jnp.float32)
        return carry

    lax.fori_loop(0, _ZROWS, zrow, 0)
    for t in range(_RPT // _ZROWS):
        pltpu.sync_copy(zbuf, acc.at[pl.ds(sid * _RPT + t * _ZROWS, _ZROWS)])
    plsc.subcore_barrier()

    base = wid * _EPT

    def chunk(i, carry):
        off = base + i * _K
        pltpu.sync_copy(r_hbm.at[pl.ds(off, _K)], r_v)
        pltpu.sync_copy(c_hbm.at[pl.ds(off, _K)], c_v)
        pltpu.sync_copy(ew_hbm.at[pl.ds(off, _K)], ew_v)
        pltpu.async_copy(g_hbm.at[r_v], rows, sem).wait()

        def scale16(jo, inner):
            j0 = jo * 16
            wv = ew_v[pl.ds(j0, 16)]
            for jj in range(16):
                w = jnp.full((16,), wv[jj], jnp.float32)
                for q in range(_H // 16):
                    rows[j0 + jj, pl.ds(q * 16, 16)] = (
                        rows[j0 + jj, pl.ds(q * 16, 16)] * w)
            return inner

        lax.fori_loop(0, _K // 16, scale16, 0)
        pltpu.sync_copy(rows, acc.at[c_v], add=True)
        return carry

    lax.fori_loop(0, _NCHUNK, chunk, 0)
    plsc.subcore_barrier()
    for t in range(_RPT // _ZROWS):
        r0 = sid * _RPT + t * _ZROWS
        pltpu.sync_copy(acc.at[pl.ds(r0, _ZROWS)],
                        out_hbm.at[cid, pl.ds(r0, _ZROWS)])


def _tc1_body(d0_ref, d1_ref, x_ref, w1_ref, dis_ref, g1_ref):
    deg = d0_ref[...] + d1_ref[...] + 1.0
    dis = jnp.where(deg > 0.0, lax.rsqrt(deg), 0.0)
    dis_ref[...] = dis
    h = jnp.dot(x_ref[...], w1_ref[...], preferred_element_type=jnp.float32)
    g1_ref[...] = h * dis


def _tc2_body(a0_ref, a1_ref, g1_ref, dis_ref, b1_ref, w2_ref,
              h1_ref, g2_ref):
    dis = dis_ref[...]
    pre = (a0_ref[...] + a1_ref[...] + g1_ref[...]) * dis + b1_ref[...]
    h1 = jnp.maximum(pre, 0.0)
    h1_ref[...] = h1
    g2_ref[...] = jnp.dot(h1, w2_ref[...],
                          preferred_element_type=jnp.float32) * dis


def _tc3_body(a0_ref, a1_ref, g2_ref, dis_ref, b2_ref, h2_ref):
    pre = ((a0_ref[...] + a1_ref[...] + g2_ref[...]) * dis_ref[...]
           + b2_ref[...])
    h2_ref[...] = jnp.maximum(pre, 0.0)


def _row_blk(shape_cols):
    return pl.BlockSpec((_ROWBLK, shape_cols), lambda i: (i, 0))


def _full_blk(rows, cols):
    return pl.BlockSpec((rows, cols), lambda i: (0, 0))


_tc1 = pl.pallas_call(
    _tc1_body,
    grid=(_GRID,),
    in_specs=[
        _row_blk(1), _row_blk(1), _row_blk(_D), _full_blk(_D, _H),
    ],
    out_specs=[_row_blk(1), _row_blk(_H)],
    out_shape=[
        jax.ShapeDtypeStruct((_N, 1), jnp.float32),
        jax.ShapeDtypeStruct((_N, _H), jnp.float32),
    ],
)

_tc2 = pl.pallas_call(
    _tc2_body,
    grid=(_GRID,),
    in_specs=[
        _row_blk(_H), _row_blk(_H), _row_blk(_H), _row_blk(1),
        _full_blk(1, _H), _full_blk(_H, _H),
    ],
    out_specs=[_row_blk(_H), _row_blk(_H)],
    out_shape=[
        jax.ShapeDtypeStruct((_N, _H), jnp.float32),
        jax.ShapeDtypeStruct((_N, _H), jnp.float32),
    ],
)

_tc3 = pl.pallas_call(
    _tc3_body,
    grid=(_GRID,),
    in_specs=[
        _row_blk(_H), _row_blk(_H), _row_blk(_H), _row_blk(1),
        _full_blk(1, _H),
    ],
    out_specs=_row_blk(_H),
    out_shape=jax.ShapeDtypeStruct((_N, _H), jnp.float32),
)


@jax.jit
def kernel(x, edge_index, edge_weights, W1, b1, W2, b2):
    row = edge_index[0]
    col = edge_index[1]
    pad = _EPAD - _E
    row_p = jnp.concatenate([row, jnp.zeros((pad,), jnp.int32)])
    col_p = jnp.concatenate([col, jnp.zeros((pad,), jnp.int32)])
    ew_p = jnp.concatenate([edge_weights, jnp.zeros((pad,), jnp.float32)])

    deg_parts = _sc_degree(col_p, ew_p)
    d0 = deg_parts[0, :_N].reshape(_N, 1)
    d1 = deg_parts[1, :_N].reshape(_N, 1)

    dis, g1 = _tc1(d0, d1, x, W1)

    acc1 = _sc_message(g1, row_p, col_p, ew_p)
    h1, g2 = _tc2(acc1[0, :_N], acc1[1, :_N], g1, dis,
                  b1.reshape(1, _H), W2)

    acc2 = _sc_message(g2, row_p, col_p, ew_p)
    h2 = _tc3(acc2[0, :_N], acc2[1, :_N], g2, dis, b2.reshape(1, _H))

    return jnp.concatenate([h1, h2], axis=-1)
